# Initial kernel scaffold; baseline (speedup 1.0000x reference)
import jax
import jax.numpy as jnp
from jax.experimental import pallas as pl


def kernel(inputs, targets):
    raise NotImplementedError("write your pallas kernel here")


# speedup vs baseline: 9.5527x; 9.5527x over previous
import functools

import jax
import jax.numpy as jnp
from jax import lax
from jax.experimental import pallas as pl
from jax.experimental.pallas import tpu as pltpu
from jax.experimental.pallas import tpu_sc as plsc

_B = 16384
_C = 1000
_NC = 2
_NS = 16
_NW = _NC * _NS
_RPW = _B // _NW
_G = 16
_NG = _RPW // _G
_NEG = -3.0e38


def _dlr_body(x_hbm, t_hbm, o_hbm, xbuf, tbuf, obuf):
    wid = lax.axis_index("c") * _NS + lax.axis_index("s")
    row0 = wid * _RPW
    pltpu.sync_copy(t_hbm.at[pl.ds(row0, _RPW)], tbuf)

    lanes = lax.broadcasted_iota(jnp.int32, (16,), 0)
    lane_base = lanes * _C
    neg = jnp.full((16,), _NEG, jnp.float32)

    def group_body(g, carry):
        base = (row0 + g * _G) * _C
        pltpu.sync_copy(x_hbm.at[pl.ds(base, _G * _C)], xbuf)

        def col_body(c, acc):
            a1, a2, a3, idx = acc
            v = plsc.load_gather(xbuf, [idx])
            l1 = jnp.minimum(a1, v)
            a1 = jnp.maximum(a1, v)
            l2 = jnp.minimum(a2, l1)
            a2 = jnp.maximum(a2, l1)
            a3 = jnp.maximum(a3, l2)
            return a1, a2, a3, idx + 1

        a1, a2, a3, _ = lax.fori_loop(0, _C, col_body,
                                      (neg, neg, neg, lane_base))
        t16 = tbuf[pl.ds(g * _G, 16)]
        tl = plsc.load_gather(xbuf, [lane_base + t16])
        num = jnp.where(tl == a1, a2, a1) - tl
        den = (a1 - a3) + jnp.float32(1e-12)
        obuf[pl.ds(g * _G, 16)] = num / den
        return carry

    lax.fori_loop(0, _NG, group_body, jnp.int32(0))
    pltpu.sync_copy(obuf, o_hbm.at[pl.ds(row0, _RPW)])


@functools.partial(
    pl.kernel,
    out_type=jax.ShapeDtypeStruct((_B,), jnp.float32),
    mesh=plsc.VectorSubcoreMesh(core_axis_name="c", subcore_axis_name="s"),
    compiler_params=pltpu.CompilerParams(needs_layout_passes=False),
    scratch_types=[
        pltpu.VMEM((_G * _C,), jnp.float32),
        pltpu.VMEM((_RPW,), jnp.int32),
        pltpu.VMEM((_RPW,), jnp.float32),
    ],
)
def _dlr_sc(x_hbm, t_hbm, o_hbm, xbuf, tbuf, obuf):
    _dlr_body(x_hbm, t_hbm, o_hbm, xbuf, tbuf, obuf)


def kernel(inputs, targets):
    x_flat = inputs.reshape(-1)
    t32 = targets.astype(jnp.int32)
    return _dlr_sc(x_flat, t32)
